# Initial kernel scaffold; baseline (speedup 1.0000x reference)
import jax
import jax.numpy as jnp
from jax.experimental import pallas as pl


def kernel(x, edge_index, batch, W0, b0, W1, b1, W2, b2, W3, b3, conv1_w, conv1_b, conv2_w, conv2_b, fc1_W, fc1_b, fc2_W, fc2_b):
    raise NotImplementedError("write your pallas kernel here")


# speedup vs baseline: 11.1908x; 11.1908x over previous
import functools

import jax
import jax.numpy as jnp
from jax import lax
from jax.experimental import pallas as pl
from jax.experimental.pallas import tpu as pltpu
from jax.experimental.pallas import tpu_sc as plsc

_N = 10000
_NP = 10240
_E = 320000
_ETOT = _E + _N
_G = 128
_K = 35
_H = 32
_D = 128

_NC = 2
_NS = 16
_NW = _NC * _NS

_CHUNK = 512
_CHUNKS = 21
_EPW = _CHUNK * _CHUNKS
_EP = _EPW * _NW
_ROWS_PER_S = _NP // _NS

_GK = _G * _K
_GKP = 4608


def _make_sc_scatter(F):
    mesh = plsc.VectorSubcoreMesh(core_axis_name="c", subcore_axis_name="s")

    @functools.partial(
        pl.kernel,
        mesh=mesh,
        compiler_params=pltpu.CompilerParams(use_tc_tiling_on_sc=False),
        out_type=jax.ShapeDtypeStruct((_NC, _NP, F), jnp.float32),
        scratch_types=[
            pltpu.VMEM((4, 128), jnp.int32),
            pltpu.VMEM((4, 128), jnp.int32),
            pltpu.VMEM((_CHUNK, F), jnp.float32),
            pltpu.VMEM_SHARED((_NP, F), jnp.float32),
            pltpu.SemaphoreType.DMA,
        ],
    )
    def k(tab_hbm, src_hbm, dst_hbm, zeros_hbm, out_hbm, src_v, dst_v, rows_v, acc_sh, sem):
        c = lax.axis_index("c")
        s = lax.axis_index("s")
        wid = s * _NC + c
        r0 = s * _ROWS_PER_S
        pltpu.sync_copy(zeros_hbm.at[pl.ds(r0, _ROWS_PER_S)],
                        acc_sh.at[pl.ds(r0, _ROWS_PER_S)])
        plsc.subcore_barrier()

        base_row = wid * (_EPW // 128)

        def body(i, carry):
            j0 = base_row + i * 4
            pltpu.sync_copy(src_hbm.at[pl.ds(j0, 4)], src_v)
            pltpu.sync_copy(dst_hbm.at[pl.ds(j0, 4)], dst_v)
            cps = [pltpu.async_copy(tab_hbm.at[src_v.at[j]],
                                    rows_v.at[pl.ds(j * 128, 128)], sem)
                   for j in range(4)]
            for cp in cps:
                cp.wait()
            for j in range(4):
                pltpu.sync_copy(rows_v.at[pl.ds(j * 128, 128)],
                                acc_sh.at[dst_v.at[j]], add=True)
            return carry

        lax.fori_loop(0, _CHUNKS, body, 0)
        plsc.subcore_barrier()
        pltpu.sync_copy(acc_sh.at[pl.ds(r0, _ROWS_PER_S)],
                        out_hbm.at[c, pl.ds(r0, _ROWS_PER_S)])

    return k


def _make_sc_gather():
    mesh = plsc.VectorSubcoreMesh(core_axis_name="c", subcore_axis_name="s")
    per_w = _GKP // _NW

    @functools.partial(
        pl.kernel,
        mesh=mesh,
        compiler_params=pltpu.CompilerParams(use_tc_tiling_on_sc=False),
        out_type=jax.ShapeDtypeStruct((_GKP, _D), jnp.float32),
        scratch_types=[
            pltpu.VMEM((per_w,), jnp.int32),
            pltpu.VMEM((per_w, _D), jnp.float32),
            pltpu.SemaphoreType.DMA,
        ],
    )
    def k(tab_hbm, idx_hbm, out_hbm, idx_v, rows_v, sem):
        c = lax.axis_index("c")
        s = lax.axis_index("s")
        wid = s * _NC + c
        base = wid * per_w
        pltpu.sync_copy(idx_hbm.at[pl.ds(base, per_w)], idx_v)
        cp0 = pltpu.async_copy(tab_hbm.at[idx_v.at[pl.ds(0, 128)]],
                               rows_v.at[pl.ds(0, 128)], sem)
        cp1 = pltpu.async_copy(tab_hbm.at[idx_v.at[pl.ds(128, 16)]],
                               rows_v.at[pl.ds(128, 16)], sem)
        cp0.wait()
        cp1.wait()
        pltpu.sync_copy(rows_v, out_hbm.at[pl.ds(base, per_w)])

    return k


def _dinv_body(dp_ref, out_ref):
    deg = dp_ref[0] + dp_ref[1]
    d0 = deg[:, 0:1]
    dinv = jnp.where(d0 > 0.0, lax.rsqrt(jnp.maximum(d0, 1e-30)), 0.0)
    out_ref[...] = jnp.broadcast_to(dinv, (_NP, 128))


def _dinv_rep(deg_parts):
    return pl.pallas_call(
        _dinv_body,
        out_shape=jax.ShapeDtypeStruct((_NP, 128), jnp.float32),
    )(deg_parts)


def _p0_body(x_ref, w_ref, dv_ref, out_ref):
    m = jnp.dot(x_ref[...], w_ref[...], preferred_element_type=jnp.float32)
    out_ref[...] = m * dv_ref[:, :_H]


def _p0(x_pad, W0, dinv_rep):
    R = 1024
    return pl.pallas_call(
        _p0_body,
        grid=(_NP // R,),
        in_specs=[
            pl.BlockSpec((R, _D), lambda r: (r, 0)),
            pl.BlockSpec((_D, _H), lambda r: (0, 0)),
            pl.BlockSpec((R, 128), lambda r: (r, 0)),
        ],
        out_specs=pl.BlockSpec((R, _H), lambda r: (r, 0)),
        out_shape=jax.ShapeDtypeStruct((_NP, _H), jnp.float32),
    )(x_pad, W0, dinv_rep)


def _layer_body(ap_ref, dv_ref, b_ref, w_ref, h_ref, p_ref):
    dv = dv_ref[:, :_H]
    agg = (ap_ref[0] + ap_ref[1]) * dv + b_ref[...]
    h = jnp.tanh(agg)
    h_ref[...] = h
    p_ref[...] = jnp.dot(h, w_ref[...], preferred_element_type=jnp.float32) * dv


def _layer_last_body(ap_ref, dv_ref, b_ref, h_ref):
    dv = dv_ref[:, :_H]
    agg = (ap_ref[0] + ap_ref[1]) * dv + b_ref[...]
    h_ref[...] = jnp.tanh(agg)


def _layer(acc_parts, dinv_rep, b_row, W_next):
    R = 1024
    return pl.pallas_call(
        _layer_body,
        grid=(_NP // R,),
        in_specs=[
            pl.BlockSpec((2, R, _H), lambda r: (0, r, 0)),
            pl.BlockSpec((R, 128), lambda r: (r, 0)),
            pl.BlockSpec((1, _H), lambda r: (0, 0)),
            pl.BlockSpec((_H, _H), lambda r: (0, 0)),
        ],
        out_specs=[
            pl.BlockSpec((R, _H), lambda r: (r, 0)),
            pl.BlockSpec((R, _H), lambda r: (r, 0)),
        ],
        out_shape=[
            jax.ShapeDtypeStruct((_NP, _H), jnp.float32),
            jax.ShapeDtypeStruct((_NP, _H), jnp.float32),
        ],
    )(acc_parts, dinv_rep, b_row, W_next)


def _layer_last(acc_parts, dinv_rep, b_row):
    R = 1024
    return pl.pallas_call(
        _layer_last_body,
        grid=(_NP // R,),
        in_specs=[
            pl.BlockSpec((2, R, _H), lambda r: (0, r, 0)),
            pl.BlockSpec((R, 128), lambda r: (r, 0)),
            pl.BlockSpec((1, _H), lambda r: (0, 0)),
        ],
        out_specs=pl.BlockSpec((R, _H), lambda r: (r, 0)),
        out_shape=jax.ShapeDtypeStruct((_NP, _H), jnp.float32),
    )(acc_parts, dinv_rep, b_row)


def _topk_body(h3_ref, batch_ref, vals_ref, idx_ref, sc_ref):
    gid = lax.broadcasted_iota(jnp.int32, (1, _G), 1)
    s = h3_ref[:, _H - 1:_H]
    mask = batch_ref[...] == gid
    neg = jnp.float32(-jnp.inf)
    sc_ref[...] = jnp.where(mask, s, neg)
    rows = lax.broadcasted_iota(jnp.int32, (_NP, _G), 0)

    def body(k, carry):
        sc = sc_ref[...]
        m = jnp.max(sc, axis=0, keepdims=True)
        cand = jnp.where(sc == m, rows, _NP)
        ii = jnp.min(cand, axis=0, keepdims=True)
        vals_ref[pl.ds(k, 1), :] = m
        idx_ref[pl.ds(k, 1), :] = ii
        sc_ref[...] = jnp.where(rows == ii, neg, sc)
        return carry

    lax.fori_loop(0, _K, body, 0)


def _topk(h3, batch_col):
    return pl.pallas_call(
        _topk_body,
        out_shape=[
            jax.ShapeDtypeStruct((40, _G), jnp.float32),
            jax.ShapeDtypeStruct((40, _G), jnp.int32),
        ],
        scratch_shapes=[pltpu.VMEM((_NP, _G), jnp.float32)],
    )(h3, batch_col)


def _tail_body(f_ref, mask_ref, c1_ref, c1b_ref, w2_ref, c2b_ref,
               fc1_ref, fc1b_ref, fc2_ref, fc2b_ref, out_ref):
    feats = f_ref[...] * mask_ref[...]
    a = jnp.dot(feats, c1_ref[...], preferred_element_type=jnp.float32)
    a = jax.nn.relu(a + c1b_ref[...])
    pooled = [jnp.maximum(a[2 * t * _G:(2 * t + 1) * _G],
                          a[(2 * t + 1) * _G:(2 * t + 2) * _G])
              for t in range(17)]
    acc1 = jnp.zeros((_G, 128), jnp.float32)
    for t in range(13):
        zt = c2b_ref[...]
        for s in range(5):
            w2s = w2_ref[16 * s:16 * s + 16, :]
            zt = zt + jnp.dot(pooled[t + s], w2s, preferred_element_type=jnp.float32)
        zt = jax.nn.relu(zt)
        fc1t = fc1_ref[32 * t:32 * t + 32, :]
        acc1 = acc1 + jnp.dot(zt, fc1t, preferred_element_type=jnp.float32)
    z = jax.nn.relu(acc1 + fc1b_ref[...])
    out = jnp.dot(z, fc2_ref[...], preferred_element_type=jnp.float32) + fc2b_ref[...]
    out_ref[...] = jax.nn.sigmoid(out)


def _tail(feats, mask_col, c1, c1b, w2u, c2b, fc1p, fc1b, fc2, fc2b):
    return pl.pallas_call(
        _tail_body,
        out_shape=jax.ShapeDtypeStruct((_G, 1), jnp.float32),
    )(feats, mask_col, c1, c1b, w2u, c2b, fc1p, fc1b, fc2, fc2b)


_get_sc_scatter = functools.lru_cache(None)(_make_sc_scatter)
_get_sc_gather = functools.lru_cache(None)(_make_sc_gather)


def kernel(x, edge_index, batch, W0, b0, W1, b1, W2, b2, W3, b3,
           conv1_w, conv1_b, conv2_w, conv2_b, fc1_W, fc1_b, fc2_W, fc2_b):
    f32 = jnp.float32
    loop = jnp.arange(_N, dtype=jnp.int32)
    src = jnp.concatenate([edge_index[0].astype(jnp.int32), loop,
                           jnp.full((_EP - _ETOT,), _N, jnp.int32)])
    dst = jnp.concatenate([edge_index[1].astype(jnp.int32), loop,
                           jnp.zeros((_EP - _ETOT,), jnp.int32)])
    src2d = src.reshape(_EP // 128, 128)
    dst2d = dst.reshape(_EP // 128, 128)

    x_pad = jnp.zeros((_NP, _D), f32).at[:_N].set(x)
    batch_col = jnp.full((_NP, 1), _G, jnp.int32).at[:_N, 0].set(batch.astype(jnp.int32))

    node_iota = jnp.arange(_NP, dtype=jnp.int32)[:, None]
    ones_tab = jnp.where(node_iota < _N, 1.0, 0.0).astype(f32) * jnp.ones((1, 16), f32)
    zeros16 = jnp.zeros((_NP, 16), f32)
    zeros32 = jnp.zeros((_NP, _H), f32)

    deg_parts = _get_sc_scatter(16)(ones_tab, src2d, dst2d, zeros16)
    dinv_rep = _dinv_rep(deg_parts)

    p = _p0(x_pad, W0, dinv_rep)
    hs = []
    Ws_next = [W1, W2, W3]
    bs = [b0, b1, b2, b3]
    for i in range(4):
        acc_parts = _get_sc_scatter(_H)(p, src2d, dst2d, zeros32)
        b_row = bs[i].reshape(1, _H).astype(f32)
        if i < 3:
            h, p = _layer(acc_parts, dinv_rep, b_row, Ws_next[i])
        else:
            h = _layer_last(acc_parts, dinv_rep, b_row)
        hs.append(h)

    hcat = jnp.concatenate(hs, axis=1)

    vals, idx = _topk(hs[3], batch_col)
    idx_flat = idx[:_K].reshape(_GK)
    idx_pad = jnp.concatenate([idx_flat, jnp.zeros((_GKP - _GK,), jnp.int32)])
    mask_col = jnp.isfinite(vals[:_K]).reshape(_GK, 1).astype(f32)

    rows = _get_sc_gather()(hcat, idx_pad)
    feats = rows[:_GK]

    c1 = conv1_w[:, :, 0].T.astype(f32)
    c1b = conv1_b.reshape(1, 16).astype(f32)
    w2u = conv2_w.transpose(2, 1, 0).reshape(80, 32).astype(f32)
    c2b = conv2_b.reshape(1, 32).astype(f32)
    fc1p = fc1_W.reshape(32, 13, 128).transpose(1, 0, 2).reshape(416, 128).astype(f32)
    fc1b = fc1_b.reshape(1, 128).astype(f32)
    fc2 = fc2_W.astype(f32)
    fc2b = fc2_b.reshape(1, 1).astype(f32)
    return _tail(feats, mask_col, c1, c1b, w2u, c2b, fc1p, fc1b, fc2, fc2b)


# speedup vs baseline: 12.9968x; 1.1614x over previous
import functools

import jax
import jax.numpy as jnp
from jax import lax
from jax.experimental import pallas as pl
from jax.experimental.pallas import tpu as pltpu
from jax.experimental.pallas import tpu_sc as plsc

_N = 10000
_NP = 10240
_E = 320000
_ETOT = _E + _N
_G = 128
_K = 35
_H = 32
_D = 128

_NC = 2
_NS = 16
_NW = _NC * _NS

_CHUNK = 512
_CHUNKS = 21
_EPW = _CHUNK * _CHUNKS
_EP = _EPW * _NW
_ROWS_PER_S = _NP // _NS

_GK = _G * _K
_GKP = 4608


def _make_sc_scatter(F):
    mesh = plsc.VectorSubcoreMesh(core_axis_name="c", subcore_axis_name="s")

    @functools.partial(
        pl.kernel,
        mesh=mesh,
        compiler_params=pltpu.CompilerParams(use_tc_tiling_on_sc=False),
        out_type=jax.ShapeDtypeStruct((_NC, _NP, F), jnp.float32),
        scratch_types=[
            pltpu.VMEM((2, 4, 128), jnp.int32),
            pltpu.VMEM((2, 4, 128), jnp.int32),
            pltpu.VMEM((2, _CHUNK, F), jnp.float32),
            pltpu.VMEM_SHARED((_NP, F), jnp.float32),
            pltpu.SemaphoreType.DMA,
            pltpu.SemaphoreType.DMA,
        ],
    )
    def k(tab_hbm, src_hbm, dst_hbm, zeros_hbm, out_hbm,
          src_v, dst_v, rows_v, acc_sh, sem_g, sem_s):
        c = lax.axis_index("c")
        s = lax.axis_index("s")
        wid = s * _NC + c
        r0 = s * _ROWS_PER_S
        pltpu.sync_copy(zeros_hbm.at[pl.ds(r0, _ROWS_PER_S)],
                        acc_sh.at[pl.ds(r0, _ROWS_PER_S)])
        plsc.subcore_barrier()

        base_row = wid * (_EPW // 128)

        def load_idx(ci, b):
            j0 = base_row + ci * 4
            pltpu.sync_copy(src_hbm.at[pl.ds(j0, 4)], src_v.at[b])
            pltpu.sync_copy(dst_hbm.at[pl.ds(j0, 4)], dst_v.at[b])

        def fire_gathers(b):
            for j in range(4):
                pltpu.async_copy(tab_hbm.at[src_v.at[b, j]],
                                 rows_v.at[b, pl.ds(j * 128, 128)], sem_g)

        def wait_gathers(b):
            for j in range(4):
                pltpu.make_async_copy(tab_hbm.at[src_v.at[b, j]],
                                      rows_v.at[b, pl.ds(j * 128, 128)],
                                      sem_g).wait()

        def scatter(b):
            cps = [pltpu.async_copy(rows_v.at[b, pl.ds(j * 128, 128)],
                                    acc_sh.at[dst_v.at[b, j]], sem_s, add=True)
                   for j in range(4)]
            for cp in cps:
                cp.wait()

        load_idx(0, 0)
        fire_gathers(0)

        def body(i, carry):
            ci = 2 * i
            load_idx(ci + 1, 1)
            fire_gathers(1)
            wait_gathers(0)
            scatter(0)
            load_idx(ci + 2, 0)
            fire_gathers(0)
            wait_gathers(1)
            scatter(1)
            return carry

        lax.fori_loop(0, (_CHUNKS - 1) // 2, body, 0)
        wait_gathers(0)
        scatter(0)

        plsc.subcore_barrier()
        pltpu.sync_copy(acc_sh.at[pl.ds(r0, _ROWS_PER_S)],
                        out_hbm.at[c, pl.ds(r0, _ROWS_PER_S)])

    return k


def _make_sc_deg():
    mesh = plsc.VectorSubcoreMesh(core_axis_name="c", subcore_axis_name="s")

    @functools.partial(
        pl.kernel,
        mesh=mesh,
        compiler_params=pltpu.CompilerParams(use_tc_tiling_on_sc=False),
        out_type=jax.ShapeDtypeStruct((_NC, _NP, 16), jnp.float32),
        scratch_types=[
            pltpu.VMEM((2, 4, 128), jnp.int32),
            pltpu.VMEM((128, 16), jnp.float32),
            pltpu.VMEM_SHARED((_NP, 16), jnp.float32),
            pltpu.SemaphoreType.DMA,
        ],
    )
    def k(dst_hbm, zeros_hbm, ones_hbm, out_hbm, dst_v, ones_v, acc_sh, sem_s):
        c = lax.axis_index("c")
        s = lax.axis_index("s")
        wid = s * _NC + c
        r0 = s * _ROWS_PER_S
        pltpu.sync_copy(zeros_hbm.at[pl.ds(r0, _ROWS_PER_S)],
                        acc_sh.at[pl.ds(r0, _ROWS_PER_S)])
        pltpu.sync_copy(ones_hbm, ones_v)
        plsc.subcore_barrier()

        base_row = wid * (_EPW // 128)

        def load_idx(ci, b):
            pltpu.sync_copy(dst_hbm.at[pl.ds(base_row + ci * 4, 4)], dst_v.at[b])

        def scatter(b):
            cps = [pltpu.async_copy(ones_v, acc_sh.at[dst_v.at[b, j]], sem_s,
                                    add=True)
                   for j in range(4)]
            for cp in cps:
                cp.wait()

        load_idx(0, 0)

        def body(i, carry):
            ci = 2 * i
            load_idx(ci + 1, 1)
            scatter(0)
            load_idx(ci + 2, 0)
            scatter(1)
            return carry

        lax.fori_loop(0, (_CHUNKS - 1) // 2, body, 0)
        scatter(0)

        plsc.subcore_barrier()
        pltpu.sync_copy(acc_sh.at[pl.ds(r0, _ROWS_PER_S)],
                        out_hbm.at[c, pl.ds(r0, _ROWS_PER_S)])

    return k


def _make_sc_gather():
    mesh = plsc.VectorSubcoreMesh(core_axis_name="c", subcore_axis_name="s")
    per_w = _GKP // _NW

    @functools.partial(
        pl.kernel,
        mesh=mesh,
        compiler_params=pltpu.CompilerParams(use_tc_tiling_on_sc=False),
        out_type=jax.ShapeDtypeStruct((_GKP, _D), jnp.float32),
        scratch_types=[
            pltpu.VMEM((per_w,), jnp.int32),
            pltpu.VMEM((per_w, _D), jnp.float32),
            pltpu.SemaphoreType.DMA,
        ],
    )
    def k(tab_hbm, idx_hbm, out_hbm, idx_v, rows_v, sem):
        c = lax.axis_index("c")
        s = lax.axis_index("s")
        wid = s * _NC + c
        base = wid * per_w
        pltpu.sync_copy(idx_hbm.at[pl.ds(base, per_w)], idx_v)
        cp0 = pltpu.async_copy(tab_hbm.at[idx_v.at[pl.ds(0, 128)]],
                               rows_v.at[pl.ds(0, 128)], sem)
        cp1 = pltpu.async_copy(tab_hbm.at[idx_v.at[pl.ds(128, 16)]],
                               rows_v.at[pl.ds(128, 16)], sem)
        cp0.wait()
        cp1.wait()
        pltpu.sync_copy(rows_v, out_hbm.at[pl.ds(base, per_w)])

    return k


def _dinv_body(dp_ref, out_ref):
    deg = dp_ref[0] + dp_ref[1]
    d0 = deg[:, 0:1]
    dinv = jnp.where(d0 > 0.0, lax.rsqrt(jnp.maximum(d0, 1e-30)), 0.0)
    out_ref[...] = jnp.broadcast_to(dinv, (_NP, 128))


def _dinv_rep(deg_parts):
    return pl.pallas_call(
        _dinv_body,
        out_shape=jax.ShapeDtypeStruct((_NP, 128), jnp.float32),
    )(deg_parts)


def _p0_body(x_ref, w_ref, dv_ref, out_ref):
    m = jnp.dot(x_ref[...], w_ref[...], preferred_element_type=jnp.float32)
    out_ref[...] = m * dv_ref[:, :_H]


def _p0(x_pad, W0, dinv_rep):
    R = 1024
    return pl.pallas_call(
        _p0_body,
        grid=(_NP // R,),
        in_specs=[
            pl.BlockSpec((R, _D), lambda r: (r, 0)),
            pl.BlockSpec((_D, _H), lambda r: (0, 0)),
            pl.BlockSpec((R, 128), lambda r: (r, 0)),
        ],
        out_specs=pl.BlockSpec((R, _H), lambda r: (r, 0)),
        out_shape=jax.ShapeDtypeStruct((_NP, _H), jnp.float32),
    )(x_pad, W0, dinv_rep)


def _layer_body(ap_ref, dv_ref, b_ref, w_ref, h_ref, p_ref):
    dv = dv_ref[:, :_H]
    agg = (ap_ref[0] + ap_ref[1]) * dv + b_ref[...]
    h = jnp.tanh(agg)
    h_ref[...] = h
    p_ref[...] = jnp.dot(h, w_ref[...], preferred_element_type=jnp.float32) * dv


def _layer_last_body(ap_ref, dv_ref, b_ref, h_ref):
    dv = dv_ref[:, :_H]
    agg = (ap_ref[0] + ap_ref[1]) * dv + b_ref[...]
    h_ref[...] = jnp.tanh(agg)


def _layer(acc_parts, dinv_rep, b_row, W_next):
    R = 1024
    return pl.pallas_call(
        _layer_body,
        grid=(_NP // R,),
        in_specs=[
            pl.BlockSpec((2, R, _H), lambda r: (0, r, 0)),
            pl.BlockSpec((R, 128), lambda r: (r, 0)),
            pl.BlockSpec((1, _H), lambda r: (0, 0)),
            pl.BlockSpec((_H, _H), lambda r: (0, 0)),
        ],
        out_specs=[
            pl.BlockSpec((R, _H), lambda r: (r, 0)),
            pl.BlockSpec((R, _H), lambda r: (r, 0)),
        ],
        out_shape=[
            jax.ShapeDtypeStruct((_NP, _H), jnp.float32),
            jax.ShapeDtypeStruct((_NP, _H), jnp.float32),
        ],
    )(acc_parts, dinv_rep, b_row, W_next)


def _layer_last(acc_parts, dinv_rep, b_row):
    R = 1024
    return pl.pallas_call(
        _layer_last_body,
        grid=(_NP // R,),
        in_specs=[
            pl.BlockSpec((2, R, _H), lambda r: (0, r, 0)),
            pl.BlockSpec((R, 128), lambda r: (r, 0)),
            pl.BlockSpec((1, _H), lambda r: (0, 0)),
        ],
        out_specs=pl.BlockSpec((R, _H), lambda r: (r, 0)),
        out_shape=jax.ShapeDtypeStruct((_NP, _H), jnp.float32),
    )(acc_parts, dinv_rep, b_row)


def _topk_body(h3_ref, batch_ref, vals_ref, idx_ref, sc_ref):
    gid = lax.broadcasted_iota(jnp.int32, (1, _G), 1)
    s = h3_ref[:, _H - 1:_H]
    mask = batch_ref[...] == gid
    neg = jnp.float32(-jnp.inf)
    sc_ref[...] = jnp.where(mask, s, neg)
    rows = lax.broadcasted_iota(jnp.int32, (_NP, _G), 0)

    def body(k, carry):
        sc = sc_ref[...]
        m = jnp.max(sc, axis=0, keepdims=True)
        cand = jnp.where(sc == m, rows, _NP)
        ii = jnp.min(cand, axis=0, keepdims=True)
        vals_ref[pl.ds(k, 1), :] = m
        idx_ref[pl.ds(k, 1), :] = ii
        sc_ref[...] = jnp.where(rows == ii, neg, sc)
        return carry

    lax.fori_loop(0, _K, body, 0)


def _topk(h3, batch_col):
    return pl.pallas_call(
        _topk_body,
        out_shape=[
            jax.ShapeDtypeStruct((40, _G), jnp.float32),
            jax.ShapeDtypeStruct((40, _G), jnp.int32),
        ],
        scratch_shapes=[pltpu.VMEM((_NP, _G), jnp.float32)],
    )(h3, batch_col)


def _tail_body(f_ref, mask_ref, c1_ref, c1b_ref, w2_ref, c2b_ref,
               fc1_ref, fc1b_ref, fc2_ref, fc2b_ref, out_ref):
    feats = f_ref[...] * mask_ref[...]
    a = jnp.dot(feats, c1_ref[...], preferred_element_type=jnp.float32)
    a = jax.nn.relu(a + c1b_ref[...])
    pooled = [jnp.maximum(a[2 * t * _G:(2 * t + 1) * _G],
                          a[(2 * t + 1) * _G:(2 * t + 2) * _G])
              for t in range(17)]
    acc1 = jnp.zeros((_G, 128), jnp.float32)
    for t in range(13):
        zt = c2b_ref[...]
        for s in range(5):
            w2s = w2_ref[16 * s:16 * s + 16, :]
            zt = zt + jnp.dot(pooled[t + s], w2s, preferred_element_type=jnp.float32)
        zt = jax.nn.relu(zt)
        fc1t = fc1_ref[32 * t:32 * t + 32, :]
        acc1 = acc1 + jnp.dot(zt, fc1t, preferred_element_type=jnp.float32)
    z = jax.nn.relu(acc1 + fc1b_ref[...])
    out = jnp.dot(z, fc2_ref[...], preferred_element_type=jnp.float32) + fc2b_ref[...]
    out_ref[...] = jax.nn.sigmoid(out)


def _tail(feats, mask_col, c1, c1b, w2u, c2b, fc1p, fc1b, fc2, fc2b):
    return pl.pallas_call(
        _tail_body,
        out_shape=jax.ShapeDtypeStruct((_G, 1), jnp.float32),
    )(feats, mask_col, c1, c1b, w2u, c2b, fc1p, fc1b, fc2, fc2b)


_get_sc_scatter = functools.lru_cache(None)(_make_sc_scatter)
_get_sc_deg = functools.lru_cache(None)(_make_sc_deg)
_get_sc_gather = functools.lru_cache(None)(_make_sc_gather)


def kernel(x, edge_index, batch, W0, b0, W1, b1, W2, b2, W3, b3,
           conv1_w, conv1_b, conv2_w, conv2_b, fc1_W, fc1_b, fc2_W, fc2_b):
    f32 = jnp.float32
    loop = jnp.arange(_N, dtype=jnp.int32)
    src = jnp.concatenate([edge_index[0].astype(jnp.int32), loop,
                           jnp.full((_EP - _ETOT,), _N, jnp.int32)])
    dst = jnp.concatenate([edge_index[1].astype(jnp.int32), loop,
                           jnp.full((_EP - _ETOT,), _NP - 1, jnp.int32)])
    src2d = src.reshape(_EP // 128, 128)
    dst2d = dst.reshape(_EP // 128, 128)

    x_pad = jnp.zeros((_NP, _D), f32).at[:_N].set(x)
    batch_col = jnp.full((_NP, 1), _G, jnp.int32).at[:_N, 0].set(batch.astype(jnp.int32))

    ones_rows = jnp.ones((128, 16), f32)
    zeros16 = jnp.zeros((_NP, 16), f32)
    zeros32 = jnp.zeros((_NP, _H), f32)

    deg_parts = _get_sc_deg()(dst2d, zeros16, ones_rows)
    dinv_rep = _dinv_rep(deg_parts)

    p = _p0(x_pad, W0, dinv_rep)
    hs = []
    Ws_next = [W1, W2, W3]
    bs = [b0, b1, b2, b3]
    for i in range(4):
        acc_parts = _get_sc_scatter(_H)(p, src2d, dst2d, zeros32)
        b_row = bs[i].reshape(1, _H).astype(f32)
        if i < 3:
            h, p = _layer(acc_parts, dinv_rep, b_row, Ws_next[i])
        else:
            h = _layer_last(acc_parts, dinv_rep, b_row)
        hs.append(h)

    hcat = jnp.concatenate(hs, axis=1)

    vals, idx = _topk(hs[3], batch_col)
    idx_flat = idx[:_K].reshape(_GK)
    idx_pad = jnp.concatenate([idx_flat, jnp.zeros((_GKP - _GK,), jnp.int32)])
    mask_col = jnp.isfinite(vals[:_K]).reshape(_GK, 1).astype(f32)

    rows = _get_sc_gather()(hcat, idx_pad)
    feats = rows[:_GK]

    c1 = conv1_w[:, :, 0].T.astype(f32)
    c1b = conv1_b.reshape(1, 16).astype(f32)
    w2u = conv2_w.transpose(2, 1, 0).reshape(80, 32).astype(f32)
    c2b = conv2_b.reshape(1, 32).astype(f32)
    fc1p = fc1_W.reshape(32, 13, 128).transpose(1, 0, 2).reshape(416, 128).astype(f32)
    fc1b = fc1_b.reshape(1, 128).astype(f32)
    fc2 = fc2_W.astype(f32)
    fc2b = fc2_b.reshape(1, 1).astype(f32)
    return _tail(feats, mask_col, c1, c1b, w2u, c2b, fc1p, fc1b, fc2, fc2b)
